# baseline (speedup 1.0000x reference)
import functools

import numpy as np
import jax
import jax.numpy as jnp
from jax import lax
from jax.experimental import pallas as pl
from jax.experimental.pallas import tpu as pltpu
from jax.experimental.pallas import tpu_sc as plsc

H = 128
W = 128
C = 256
POOLSZ = 7
NPIX = POOLSZ * POOLSZ
NBOX = 2000
NWORKER = 32
BPW = 64
GLEN = 104

_GATHER_DN = lax.GatherDimensionNumbers(
    offset_dims=(), collapsed_slice_dims=(0,), start_index_map=(0,))


def _take(vec, tab):
    return lax.gather(vec, tab[:, None], _GATHER_DN, slice_sizes=(1,),
                      mode=lax.GatherScatterMode.PROMISE_IN_BOUNDS)


def _roi_body(fm_hbm, rpn_hbm, out_hbm, boxes_v,
              idx0a, idx1a, wt0a, wt1a, rows0a, rows1a,
              idx0b, idx1b, wt0b, wt1b, rows0b, rows1b,
              out_v,
              sem0a, sem1a, sem0b, sem1b, semo):
    wid = lax.axis_index("s") * 2 + lax.axis_index("c")
    base_box = wid * BPW
    nvalid = jnp.minimum(BPW, NBOX - base_box)
    pltpu.sync_copy(rpn_hbm.at[pl.ds(base_box * 4, BPW * 4)],
                    boxes_v.at[pl.ds(0, BPW * 4)])

    bufs_a = (idx0a, idx1a, wt0a, wt1a, rows0a, rows1a, sem0a, sem1a)
    bufs_b = (idx0b, idx1b, wt0b, wt1b, rows0b, rows1b, sem0b, sem1b)

    def build_and_issue(i, bufs):
        idx0, idx1, wt0, wt1, rows0, rows1, sem0, sem1 = bufs
        box_id = base_box + i
        win = boxes_v[pl.ds(4 * i, 16)]
        base_row = (box_id // 1000) * (H * W)
        lanevec = lax.iota(jnp.int32, 16)
        pos = lax.convert_element_type(lanevec & 7, jnp.float32)
        is_x = lanevec >= 8
        c1 = _take(win, jnp.where(is_x, 0, 1))
        c2 = _take(win, jnp.where(is_x, 2, 3))
        scale = (c2 - c1) * (float(H - 1) / float(POOLSZ - 1))
        inx = c1 * float(H - 1) + pos * scale
        validf = jnp.where((inx >= 0.0) & (inx <= float(H - 1)), 1.0, 0.0)
        inc = jnp.clip(inx, 0.0, float(H - 1))
        t = inc.astype(jnp.int32)
        lerp = inc - t.astype(jnp.float32)
        bt = jnp.minimum(t + 1, H - 1)
        for v in range(7):
            off = 16 * v if v < 6 else GLEN - 16
            local = lanevec + off
            in98 = local < 98
            l49 = lax.shift_right_logical(local * 1338, 16)
            p = jnp.where(in98, local - 49 * l49, 0)
            pd7 = lax.shift_right_logical(p * 9363, 16)
            ytab = jnp.where(in98, pd7, 7)
            xtab = jnp.where(in98, p - 7 * pd7, 7) + 8
            right = (local >= 49) & (local < 98)
            padm = jnp.where(in98, 1.0, 0.0)
            xv = jnp.where(right, _take(bt, xtab), _take(t, xtab))
            sl = pl.ds(off, 16)
            idx0[sl] = base_row + _take(t, ytab) * W + xv
            idx1[sl] = base_row + _take(bt, ytab) * W + xv
            ly = _take(lerp, ytab)
            lx = _take(lerp, xtab)
            wx = jnp.where(right, lx, 1.0 - lx)
            wxm = wx * (_take(validf, ytab) * _take(validf, xtab)) * padm
            wt0[sl] = (1.0 - ly) * wxm
            wt1[sl] = ly * wxm
        pltpu.async_copy(fm_hbm.at[idx0], rows0, sem0)
        pltpu.async_copy(fm_hbm.at[idx1], rows1, sem1)

    def step(i, bufs_cur, bufs_nxt):
        idx0, idx1, wt0, wt1, rows0, rows1, sem0, sem1 = bufs_cur
        box_id = base_box + i

        @pl.when(i < nvalid)
        def _wait():
            pltpu.make_async_copy(fm_hbm.at[idx0], rows0, sem0).wait()
            pltpu.make_async_copy(fm_hbm.at[idx1], rows1, sem1).wait()

        @pl.when(i + 1 < nvalid)
        def _prefetch():
            build_and_issue(i + 1, bufs_nxt)

        @pl.when(i < nvalid)
        def _compute():
            @pl.when(i >= 1)
            def _drain_prev_out():
                pltpu.make_async_copy(out_v, out_hbm.at[box_id], semo).wait()

            lanes0 = lax.iota(jnp.int32, 16) * 0

            def _unpk(ref, p, sl):
                w = lax.bitcast_convert_type(ref[p, sl], jnp.int32)
                a = lax.bitcast_convert_type(lax.shift_left(w, 16),
                                             jnp.float32)
                b = lax.bitcast_convert_type(w & jnp.int32(-65536),
                                             jnp.float32)
                return a, b

            def emit_pixel(p, w_tl, w_tr, w_bl, w_br):
                for g in range(C // 32):
                    sl = pl.ds(16 * g, 16)
                    a_tl, b_tl = _unpk(rows0, p, sl)
                    a_tr, b_tr = _unpk(rows0, 49 + p, sl)
                    a_bl, b_bl = _unpk(rows1, p, sl)
                    a_br, b_br = _unpk(rows1, 49 + p, sl)
                    out_v[2 * p, sl] = (w_tl * a_tl + w_tr * a_tr
                                        + w_bl * a_bl + w_br * a_br)
                    out_v[2 * p + 1, sl] = (w_tl * b_tl + w_tr * b_tr
                                            + w_bl * b_bl + w_br * b_br)

            def group_body(g, carry):
                base_p = 8 * g
                w0a = wt0[pl.ds(base_p, 16)]
                w0b = wt0[pl.ds(49 + base_p, 16)]
                w1a = wt1[pl.ds(base_p, 16)]
                w1b = wt1[pl.ds(49 + base_p, 16)]
                for j in range(8):
                    spl = lanes0 + j
                    emit_pixel(base_p + j,
                               _take(w0a, spl), _take(w0b, spl),
                               _take(w1a, spl), _take(w1b, spl))
                return carry

            lax.fori_loop(0, 6, group_body, 0)
            w0a = wt0[pl.ds(48, 16)]
            w0b = wt0[pl.ds(97, 16)]
            w1a = wt1[pl.ds(48, 16)]
            w1b = wt1[pl.ds(97, 16)]
            emit_pixel(48, _take(w0a, lanes0), _take(w0b, lanes0),
                       _take(w1a, lanes0), _take(w1b, lanes0))
            pltpu.async_copy(out_v, out_hbm.at[box_id], semo)

    @pl.when(0 < nvalid)
    def _prologue():
        build_and_issue(0, bufs_a)

    def pair_body(k, carry):
        step(2 * k, bufs_a, bufs_b)
        step(2 * k + 1, bufs_b, bufs_a)
        return carry

    lax.fori_loop(0, BPW // 2, pair_body, 0)

    @pl.when(0 < nvalid)
    def _drain_last_out():
        pltpu.make_async_copy(
            out_v, out_hbm.at[base_box + nvalid - 1], semo).wait()


@jax.jit
def _roi_align(fm_flat, rpn_pad):
    mesh = plsc.VectorSubcoreMesh(core_axis_name="c", subcore_axis_name="s")
    run = functools.partial(
        pl.kernel,
        out_type=jax.ShapeDtypeStruct((NBOX, 2 * NPIX, C // 2), jnp.float32),
        mesh=mesh,
        scratch_types=[
            pltpu.VMEM((BPW * 4 + 16,), jnp.float32),
            pltpu.VMEM((GLEN,), jnp.int32),
            pltpu.VMEM((GLEN,), jnp.int32),
            pltpu.VMEM((GLEN + 16,), jnp.float32),
            pltpu.VMEM((GLEN + 16,), jnp.float32),
            pltpu.VMEM((GLEN, C // 2), jnp.float32),
            pltpu.VMEM((GLEN, C // 2), jnp.float32),
            pltpu.VMEM((GLEN,), jnp.int32),
            pltpu.VMEM((GLEN,), jnp.int32),
            pltpu.VMEM((GLEN + 16,), jnp.float32),
            pltpu.VMEM((GLEN + 16,), jnp.float32),
            pltpu.VMEM((GLEN, C // 2), jnp.float32),
            pltpu.VMEM((GLEN, C // 2), jnp.float32),
            pltpu.VMEM((2 * NPIX, C // 2), jnp.float32),
            pltpu.SemaphoreType.DMA,
            pltpu.SemaphoreType.DMA,
            pltpu.SemaphoreType.DMA,
            pltpu.SemaphoreType.DMA,
            pltpu.SemaphoreType.DMA,
        ],
    )(_roi_body)
    return run(fm_flat, rpn_pad)


def kernel(feature_map, rpn_pred):
    fm_bf = feature_map.astype(jnp.bfloat16)
    packed = jnp.stack([fm_bf[..., :C // 2], fm_bf[..., C // 2:]], axis=-1)
    fm_flat = lax.bitcast_convert_type(packed, jnp.float32).reshape(
        2 * H * W, C // 2)
    rpn_flat = rpn_pred.reshape(NBOX * 4)
    rpn_pad = jnp.pad(rpn_flat, (0, (NWORKER * BPW - NBOX) * 4))
    out = _roi_align(fm_flat, rpn_pad)
    return out.reshape(2, 1000, POOLSZ, POOLSZ, C)


# speedup vs baseline: 1.9351x; 1.9351x over previous
import functools

import numpy as np
import jax
import jax.numpy as jnp
from jax import lax
from jax.experimental import pallas as pl
from jax.experimental.pallas import tpu as pltpu
from jax.experimental.pallas import tpu_sc as plsc

H = 128
W = 128
C = 256
POOLSZ = 7
NPIX = POOLSZ * POOLSZ
NBOX = 2000
NWORKER = 32
BPW = 64
GLEN = 104

_GATHER_DN = lax.GatherDimensionNumbers(
    offset_dims=(), collapsed_slice_dims=(0,), start_index_map=(0,))


def _take(vec, tab):
    return lax.gather(vec, tab[:, None], _GATHER_DN, slice_sizes=(1,),
                      mode=lax.GatherScatterMode.PROMISE_IN_BOUNDS)


def _roi_body(fm_hbm, rpn_hbm, out_hbm, boxes_v,
              idx0a, idx1a, wt0a, wt1a, rows0a, rows1a,
              idx0b, idx1b, wt0b, wt1b, rows0b, rows1b,
              out_v,
              sem0a, sem1a, sem0b, sem1b, semo):
    wid = lax.axis_index("s") * 2 + lax.axis_index("c")
    base_box = wid * BPW
    nvalid = jnp.minimum(BPW, NBOX - base_box)
    pltpu.sync_copy(rpn_hbm.at[pl.ds(base_box * 4, BPW * 4)],
                    boxes_v.at[pl.ds(0, BPW * 4)])

    bufs_a = (idx0a, idx1a, wt0a, wt1a, rows0a, rows1a, sem0a, sem1a)
    bufs_b = (idx0b, idx1b, wt0b, wt1b, rows0b, rows1b, sem0b, sem1b)

    def build_and_issue(i, bufs):
        idx0, idx1, wt0, wt1, rows0, rows1, sem0, sem1 = bufs
        box_id = base_box + i
        win = boxes_v[pl.ds(4 * i, 16)]
        base_row = (box_id // 1000) * (H * W)
        lanevec = lax.iota(jnp.int32, 16)
        pos = lax.convert_element_type(lanevec & 7, jnp.float32)
        is_x = lanevec >= 8
        c1 = _take(win, jnp.where(is_x, 0, 1))
        c2 = _take(win, jnp.where(is_x, 2, 3))
        scale = (c2 - c1) * (float(H - 1) / float(POOLSZ - 1))
        inx = c1 * float(H - 1) + pos * scale
        validf = jnp.where((inx >= 0.0) & (inx <= float(H - 1)), 1.0, 0.0)
        inc = jnp.clip(inx, 0.0, float(H - 1))
        t = inc.astype(jnp.int32)
        lerp = inc - t.astype(jnp.float32)
        bt = jnp.minimum(t + 1, H - 1)
        for v in range(7):
            off = 16 * v if v < 6 else GLEN - 16
            local = lanevec + off
            in98 = local < 98
            l49 = lax.shift_right_logical(local * 1338, 16)
            p = jnp.where(in98, local - 49 * l49, 0)
            pd7 = lax.shift_right_logical(p * 9363, 16)
            ytab = jnp.where(in98, pd7, 7)
            xtab = jnp.where(in98, p - 7 * pd7, 7) + 8
            right = (local >= 49) & (local < 98)
            padm = jnp.where(in98, 1.0, 0.0)
            xv = jnp.where(right, _take(bt, xtab), _take(t, xtab))
            sl = pl.ds(off, 16)
            idx0[sl] = base_row + _take(t, ytab) * W + xv
            idx1[sl] = base_row + _take(bt, ytab) * W + xv
            ly = _take(lerp, ytab)
            lx = _take(lerp, xtab)
            wx = jnp.where(right, lx, 1.0 - lx)
            wxm = wx * (_take(validf, ytab) * _take(validf, xtab)) * padm
            wt0[sl] = (1.0 - ly) * wxm
            wt1[sl] = ly * wxm
        pltpu.async_copy(fm_hbm.at[idx0], rows0, sem0)
        pltpu.async_copy(fm_hbm.at[idx1], rows1, sem1)

    def step(i, bufs_cur, bufs_nxt):
        idx0, idx1, wt0, wt1, rows0, rows1, sem0, sem1 = bufs_cur
        box_id = base_box + i

        @pl.when(i < nvalid)
        def _wait():
            pltpu.make_async_copy(fm_hbm.at[idx0], rows0, sem0).wait()
            pltpu.make_async_copy(fm_hbm.at[idx1], rows1, sem1).wait()

        @pl.when(i + 1 < nvalid)
        def _prefetch():
            build_and_issue(i + 1, bufs_nxt)

        @pl.when(i < nvalid)
        def _compute():
            b_img = box_id // 1000
            n_box = box_id - b_img * 1000

            @pl.when(i >= 1)
            def _drain_prev_out():
                pltpu.make_async_copy(
                    out_v, out_hbm.at[b_img, n_box], semo).wait()

            lanes0 = lax.iota(jnp.int32, 16) * 0

            def _unpk(ref, p, sl):
                w = lax.bitcast_convert_type(ref[p, sl], jnp.int32)
                a = lax.bitcast_convert_type(lax.shift_left(w, 16),
                                             jnp.float32)
                b = lax.bitcast_convert_type(w & jnp.int32(-65536),
                                             jnp.float32)
                return a, b

            def emit_pixel(p, iy, jx, w_tl, w_tr, w_bl, w_br):
                for g in range(C // 32):
                    sl = pl.ds(16 * g, 16)
                    a_tl, b_tl = _unpk(rows0, p, sl)
                    a_tr, b_tr = _unpk(rows0, 49 + p, sl)
                    a_bl, b_bl = _unpk(rows1, p, sl)
                    a_br, b_br = _unpk(rows1, 49 + p, sl)
                    out_v[iy, jx, sl] = (w_tl * a_tl + w_tr * a_tr
                                         + w_bl * a_bl + w_br * a_br)
                    out_v[iy, jx, pl.ds(128 + 16 * g, 16)] = (
                        w_tl * b_tl + w_tr * b_tr
                        + w_bl * b_bl + w_br * b_br)

            def group_body(iy, carry):
                base_p = 7 * iy
                w0a = wt0[pl.ds(base_p, 16)]
                w0b = wt0[pl.ds(49 + base_p, 16)]
                w1a = wt1[pl.ds(base_p, 16)]
                w1b = wt1[pl.ds(49 + base_p, 16)]
                for j in range(POOLSZ):
                    spl = lanes0 + j
                    emit_pixel(base_p + j, iy, j,
                               _take(w0a, spl), _take(w0b, spl),
                               _take(w1a, spl), _take(w1b, spl))
                return carry

            lax.fori_loop(0, POOLSZ, group_body, 0)
            pltpu.async_copy(out_v, out_hbm.at[b_img, n_box], semo)

    @pl.when(0 < nvalid)
    def _prologue():
        build_and_issue(0, bufs_a)

    def pair_body(k, carry):
        step(2 * k, bufs_a, bufs_b)
        step(2 * k + 1, bufs_b, bufs_a)
        return carry

    lax.fori_loop(0, BPW // 2, pair_body, 0)

    @pl.when(0 < nvalid)
    def _drain_last_out():
        last = base_box + nvalid - 1
        pltpu.make_async_copy(
            out_v, out_hbm.at[last // 1000, last % 1000], semo).wait()


@jax.jit
def _roi_align(fm_flat, rpn_pad):
    mesh = plsc.VectorSubcoreMesh(core_axis_name="c", subcore_axis_name="s")
    run = functools.partial(
        pl.kernel,
        out_type=jax.ShapeDtypeStruct((2, 1000, POOLSZ, POOLSZ, C),
                                      jnp.float32),
        mesh=mesh,
        scratch_types=[
            pltpu.VMEM((BPW * 4 + 16,), jnp.float32),
            pltpu.VMEM((GLEN,), jnp.int32),
            pltpu.VMEM((GLEN,), jnp.int32),
            pltpu.VMEM((GLEN + 16,), jnp.float32),
            pltpu.VMEM((GLEN + 16,), jnp.float32),
            pltpu.VMEM((GLEN, C // 2), jnp.float32),
            pltpu.VMEM((GLEN, C // 2), jnp.float32),
            pltpu.VMEM((GLEN,), jnp.int32),
            pltpu.VMEM((GLEN,), jnp.int32),
            pltpu.VMEM((GLEN + 16,), jnp.float32),
            pltpu.VMEM((GLEN + 16,), jnp.float32),
            pltpu.VMEM((GLEN, C // 2), jnp.float32),
            pltpu.VMEM((GLEN, C // 2), jnp.float32),
            pltpu.VMEM((POOLSZ, POOLSZ, C), jnp.float32),
            pltpu.SemaphoreType.DMA,
            pltpu.SemaphoreType.DMA,
            pltpu.SemaphoreType.DMA,
            pltpu.SemaphoreType.DMA,
            pltpu.SemaphoreType.DMA,
        ],
    )(_roi_body)
    return run(fm_flat, rpn_pad)


def kernel(feature_map, rpn_pred):
    fm_bf = feature_map.astype(jnp.bfloat16)
    packed = jnp.stack([fm_bf[..., :C // 2], fm_bf[..., C // 2:]], axis=-1)
    fm_flat = lax.bitcast_convert_type(packed, jnp.float32).reshape(
        2 * H * W, C // 2)
    rpn_flat = rpn_pred.reshape(NBOX * 4)
    rpn_pad = jnp.pad(rpn_flat, (0, (NWORKER * BPW - NBOX) * 4))
    return _roi_align(fm_flat, rpn_pad)
